# baseline (speedup 1.0000x reference)
import functools

import jax
import jax.numpy as jnp
from jax import lax
from jax.experimental import pallas as pl
from jax.experimental.pallas import tpu as pltpu
from jax.experimental.pallas import tpu_sc as plsc

_BATCH = 16384
_SEQ = 200
_D = 32
_DP = 16
_NCLS = 10
_VOCAB = 1000000

_NW = 32
_ROWS_PER_W = _BATCH // _NW
_CH = 8
_NCHUNK = _ROWS_PER_W // _CH
_IDXROW = 100
_NSTREAM = (_CH * _SEQ) // _IDXROW

_VB = 32768
_VGRID = -(-_VOCAB // _VB)


def _tc_project(tableT, Wp):
    def body(t_ref, w_ref, out_ref):
        out_ref[...] = lax.dot_general(
            t_ref[...], w_ref[...], (((0,), (1,)), ((), ())),
            preferred_element_type=jnp.float32)

    return pl.pallas_call(
        body,
        grid=(_VGRID,),
        in_specs=[
            pl.BlockSpec((_D, _VB), lambda i: (0, i)),
            pl.BlockSpec((_DP, _D), lambda i: (0, 0)),
        ],
        out_specs=pl.BlockSpec((_VB, _DP), lambda i: (i, 0)),
        out_shape=jax.ShapeDtypeStruct((_VOCAB, _DP), jnp.float32),
    )(tableT, Wp)


def _sc_gather_sum(ids2d, proj):
    mesh = plsc.VectorSubcoreMesh(core_axis_name="c", subcore_axis_name="s")
    nchunk2 = _NCHUNK // 2

    @functools.partial(
        pl.kernel,
        mesh=mesh,
        out_type=jax.ShapeDtypeStruct((_BATCH, _DP), jnp.float32),
        scratch_types=[
            pltpu.VMEM((_NSTREAM, _IDXROW), jnp.int32),
            pltpu.VMEM((_NSTREAM, _IDXROW), jnp.int32),
            pltpu.VMEM((_CH * _SEQ, _DP), jnp.float32),
            pltpu.VMEM((_CH * _SEQ, _DP), jnp.float32),
            pltpu.VMEM((_CH, _DP), jnp.float32),
            pltpu.VMEM((_CH, _DP), jnp.float32),
            pltpu.SemaphoreType.DMA,
            pltpu.SemaphoreType.DMA,
            pltpu.SemaphoreType.DMA,
            pltpu.SemaphoreType.DMA,
            pltpu.SemaphoreType.DMA,
            pltpu.SemaphoreType.DMA,
        ],
        compiler_params=pltpu.CompilerParams(use_tc_tiling_on_sc=False),
    )
    def k(ids_hbm, proj_hbm, out_hbm, idx0, idx1, rows0, rows1,
          acc0, acc1, sg0, sg1, si0, si1, ss0, ss1):
        wid = lax.axis_index("s") * 2 + lax.axis_index("c")
        base_row = wid * _ROWS_PER_W
        base_irow = wid * (_ROWS_PER_W * _SEQ // _IDXROW)

        def ids_slice(c):
            return ids_hbm.at[pl.ds(base_irow + c * _NSTREAM, _NSTREAM)]

        def out_slice(c):
            return out_hbm.at[pl.ds(base_row + c * _CH, _CH)]

        def fire_gathers(idxb, rowsb, sem):
            for j in range(_NSTREAM):
                pltpu.async_copy(
                    proj_hbm.at[idxb.at[j]],
                    rowsb.at[pl.ds(j * _IDXROW, _IDXROW)], sem)

        def drain_gathers(rowsb, sem):
            pltpu.make_async_copy(
                proj_hbm.at[pl.ds(0, _CH * _SEQ)], rowsb, sem).wait()

        def compute(rowsb, accb):
            def row_body(r, c2):
                zero = jnp.zeros((16,), jnp.float32)
                accs = (zero,) * 8

                def s_body(so, a):
                    a = list(a)
                    for u in range(8):
                        e = r * _SEQ + so * 8 + u
                        a[u] = a[u] + rowsb[e, pl.ds(0, 16)]
                    return tuple(a)

                accs = lax.fori_loop(0, _SEQ // 8, s_body, accs)
                accb[r, pl.ds(0, 16)] = (
                    ((accs[0] + accs[1]) + (accs[2] + accs[3]))
                    + ((accs[4] + accs[5]) + (accs[6] + accs[7])))
                return c2

            lax.fori_loop(0, _CH, row_body, 0)

        pltpu.sync_copy(ids_slice(0), idx0)
        fire_gathers(idx0, rows0, sg0)
        pltpu.async_copy(ids_slice(1), idx1, si1)

        def loop_body(ci2, carry):
            c0 = ci2 * 2
            not_last = ci2 < nchunk2 - 1

            pltpu.make_async_copy(ids_slice(c0 + 1), idx1, si1).wait()
            fire_gathers(idx1, rows1, sg1)
            drain_gathers(rows0, sg0)

            @pl.when(not_last)
            def _():
                pltpu.async_copy(ids_slice(c0 + 2), idx0, si0)

            @pl.when(ci2 > 0)
            def _():
                pltpu.make_async_copy(acc0, out_slice(c0), ss0).wait()

            compute(rows0, acc0)
            pltpu.async_copy(acc0, out_slice(c0), ss0)

            @pl.when(not_last)
            def _():
                pltpu.make_async_copy(ids_slice(c0 + 2), idx0, si0).wait()
                fire_gathers(idx0, rows0, sg0)

            drain_gathers(rows1, sg1)

            @pl.when(not_last)
            def _():
                pltpu.async_copy(ids_slice(c0 + 3), idx1, si1)

            @pl.when(ci2 > 0)
            def _():
                pltpu.make_async_copy(acc1, out_slice(c0 + 1), ss1).wait()

            compute(rows1, acc1)
            pltpu.async_copy(acc1, out_slice(c0 + 1), ss1)
            return carry

        lax.fori_loop(0, nchunk2, loop_body, 0)
        pltpu.make_async_copy(acc0, out_slice(0), ss0).wait()
        pltpu.make_async_copy(acc1, out_slice(1), ss1).wait()

    return k(ids2d, proj)


def _tc_epilogue(input_ids, psum, b2d):
    TB = 512

    def body(ids_ref, ps_ref, b_ref, out_ref):
        cnt = jnp.sum((ids_ref[...] != 0).astype(jnp.float32), axis=1,
                      keepdims=True)
        avg = ps_ref[...] / jnp.maximum(cnt, 1.0)
        out_ref[...] = avg[:, :_NCLS] + b_ref[...]

    return pl.pallas_call(
        body,
        grid=(_BATCH // TB,),
        in_specs=[
            pl.BlockSpec((TB, _SEQ), lambda i: (i, 0)),
            pl.BlockSpec((TB, _DP), lambda i: (i, 0)),
            pl.BlockSpec((1, _NCLS), lambda i: (0, 0)),
        ],
        out_specs=pl.BlockSpec((TB, _NCLS), lambda i: (i, 0)),
        out_shape=jax.ShapeDtypeStruct((_BATCH, _NCLS), jnp.float32),
    )(input_ids, psum, b2d)


def kernel(input_ids, table, W, b):
    ids = input_ids.astype(jnp.int32)
    Wp = jnp.zeros((_DP, _D), jnp.float32).at[:_NCLS].set(W)
    proj = _tc_project(table.T, Wp)
    ids2d = ids.reshape(_BATCH * _SEQ // _IDXROW, _IDXROW)
    psum = _sc_gather_sum(ids2d, proj)
    return _tc_epilogue(ids, psum, b.reshape(1, _NCLS))


# speedup vs baseline: 1.4321x; 1.4321x over previous
import functools

import jax
import jax.numpy as jnp
from jax import lax
from jax.experimental import pallas as pl
from jax.experimental.pallas import tpu as pltpu
from jax.experimental.pallas import tpu_sc as plsc

_BATCH = 16384
_SEQ = 200
_D = 32
_DP = 16
_NCLS = 10
_VOCAB = 1000000

_NW = 32
_ROWS_PER_W = _BATCH // _NW
_CH = 8
_NCHUNK = _ROWS_PER_W // _CH
_IDXROW = 100
_NSTREAM = (_CH * _SEQ) // _IDXROW

_VB = 32768
_VGRID = -(-_VOCAB // _VB)


def _tc_project(tableT, Wp):
    def body(t_ref, w_ref, out_ref, p_ref):
        p_ref[...] = lax.dot_general(
            t_ref[...], w_ref[...], (((0,), (1,)), ((), ())),
            preferred_element_type=jnp.float32)
        parts = [p_ref[pl.Slice(u, _VB // 8, 8), :] for u in range(8)]
        out_ref[...] = jnp.concatenate(parts, axis=1)

    return pl.pallas_call(
        body,
        grid=(_VGRID,),
        in_specs=[
            pl.BlockSpec((_D, _VB), lambda i: (0, i)),
            pl.BlockSpec((_DP, _D), lambda i: (0, 0)),
        ],
        out_specs=pl.BlockSpec((_VB // 8, 128), lambda i: (i, 0)),
        out_shape=jax.ShapeDtypeStruct((_VOCAB // 8, 128), jnp.float32),
        scratch_shapes=[pltpu.VMEM((_VB, _DP), jnp.float32)],
    )(tableT, Wp)


def _sc_gather_sum(ids2d, proj):
    mesh = plsc.VectorSubcoreMesh(core_axis_name="c", subcore_axis_name="s")
    nchunk2 = _NCHUNK // 2

    @functools.partial(
        pl.kernel,
        mesh=mesh,
        out_type=jax.ShapeDtypeStruct((_BATCH, _DP), jnp.float32),
        scratch_types=[
            pltpu.VMEM((_NSTREAM, _IDXROW), jnp.int32),
            pltpu.VMEM((_NSTREAM, _IDXROW), jnp.int32),
            pltpu.VMEM((_CH * _SEQ, _DP), jnp.float32),
            pltpu.VMEM((_CH * _SEQ, _DP), jnp.float32),
            pltpu.VMEM((_CH, _DP), jnp.float32),
            pltpu.VMEM((_CH, _DP), jnp.float32),
            pltpu.SemaphoreType.DMA,
            pltpu.SemaphoreType.DMA,
            pltpu.SemaphoreType.DMA,
            pltpu.SemaphoreType.DMA,
            pltpu.SemaphoreType.DMA,
            pltpu.SemaphoreType.DMA,
        ],
        compiler_params=pltpu.CompilerParams(use_tc_tiling_on_sc=False),
    )
    def k(ids_hbm, proj_hbm, out_hbm, idx0, idx1, rows0, rows1,
          acc0, acc1, sg0, sg1, si0, si1, ss0, ss1):
        wid = lax.axis_index("s") * 2 + lax.axis_index("c")
        base_row = wid * _ROWS_PER_W
        base_irow = wid * (_ROWS_PER_W * _SEQ // _IDXROW)

        def ids_slice(c):
            return ids_hbm.at[pl.ds(base_irow + c * _NSTREAM, _NSTREAM)]

        def out_slice(c):
            return out_hbm.at[pl.ds(base_row + c * _CH, _CH)]

        def fire_gathers(idxb, rowsb, sem):
            for j in range(_NSTREAM):
                pltpu.async_copy(
                    proj_hbm.at[idxb.at[j]],
                    rowsb.at[pl.ds(j * _IDXROW, _IDXROW)], sem)

        def drain_gathers(rowsb, sem):
            pltpu.make_async_copy(
                proj_hbm.at[pl.ds(0, _CH * _SEQ)], rowsb, sem).wait()

        def compute(rowsb, accb):
            def row_body(r, c2):
                zero = jnp.zeros((16,), jnp.float32)
                accs = (zero,) * 8

                def s_body(so, a):
                    a = list(a)
                    for u in range(8):
                        e = r * _SEQ + so * 8 + u
                        a[u] = a[u] + rowsb[e, pl.ds(0, 16)]
                    return tuple(a)

                accs = lax.fori_loop(0, _SEQ // 8, s_body, accs)
                accb[r, pl.ds(0, 16)] = (
                    ((accs[0] + accs[1]) + (accs[2] + accs[3]))
                    + ((accs[4] + accs[5]) + (accs[6] + accs[7])))
                return c2

            lax.fori_loop(0, _CH, row_body, 0)

        pltpu.sync_copy(ids_slice(0), idx0)
        fire_gathers(idx0, rows0, sg0)
        pltpu.async_copy(ids_slice(1), idx1, si1)

        def loop_body(ci2, carry):
            c0 = ci2 * 2
            not_last = ci2 < nchunk2 - 1

            pltpu.make_async_copy(ids_slice(c0 + 1), idx1, si1).wait()
            fire_gathers(idx1, rows1, sg1)
            drain_gathers(rows0, sg0)

            @pl.when(not_last)
            def _():
                pltpu.async_copy(ids_slice(c0 + 2), idx0, si0)

            @pl.when(ci2 > 0)
            def _():
                pltpu.make_async_copy(acc0, out_slice(c0), ss0).wait()

            compute(rows0, acc0)
            pltpu.async_copy(acc0, out_slice(c0), ss0)

            @pl.when(not_last)
            def _():
                pltpu.make_async_copy(ids_slice(c0 + 2), idx0, si0).wait()
                fire_gathers(idx0, rows0, sg0)

            drain_gathers(rows1, sg1)

            @pl.when(not_last)
            def _():
                pltpu.async_copy(ids_slice(c0 + 3), idx1, si1)

            @pl.when(ci2 > 0)
            def _():
                pltpu.make_async_copy(acc1, out_slice(c0 + 1), ss1).wait()

            compute(rows1, acc1)
            pltpu.async_copy(acc1, out_slice(c0 + 1), ss1)
            return carry

        lax.fori_loop(0, nchunk2, loop_body, 0)
        pltpu.make_async_copy(acc0, out_slice(0), ss0).wait()
        pltpu.make_async_copy(acc1, out_slice(1), ss1).wait()

    return k(ids2d, proj)


def _tc_counts(input_ids):
    TB = 512

    def body(ids_ref, out_ref):
        cnt = jnp.sum((ids_ref[...] != 0).astype(jnp.float32), axis=1,
                      keepdims=True)
        out_ref[...] = 1.0 / jnp.maximum(cnt, 1.0)

    return pl.pallas_call(
        body,
        grid=(_BATCH // TB,),
        in_specs=[pl.BlockSpec((TB, _SEQ), lambda i: (i, 0))],
        out_specs=pl.BlockSpec((TB, 1), lambda i: (i, 0)),
        out_shape=jax.ShapeDtypeStruct((_BATCH, 1), jnp.float32),
    )(input_ids)


def _tc_epilogue(rcnt, psum, b2d):
    TB = 2048

    def body(rc_ref, ps_ref, b_ref, out_ref):
        out_ref[...] = ps_ref[...][:, :_NCLS] * rc_ref[...] + b_ref[...]

    return pl.pallas_call(
        body,
        grid=(_BATCH // TB,),
        in_specs=[
            pl.BlockSpec((TB, 1), lambda i: (i, 0)),
            pl.BlockSpec((TB, _DP), lambda i: (i, 0)),
            pl.BlockSpec((1, _NCLS), lambda i: (0, 0)),
        ],
        out_specs=pl.BlockSpec((TB, _NCLS), lambda i: (i, 0)),
        out_shape=jax.ShapeDtypeStruct((_BATCH, _NCLS), jnp.float32),
    )(rcnt, psum, b2d)


def kernel(input_ids, table, W, b):
    ids = input_ids.astype(jnp.int32)
    Wp = jnp.zeros((_DP, _D), jnp.float32).at[:_NCLS].set(W)
    proj = _tc_project(table.T, Wp).reshape(_VOCAB, _DP)
    ids2d = ids.reshape(_BATCH * _SEQ // _IDXROW, _IDXROW)
    psum = _sc_gather_sum(ids2d, proj)
    rcnt = _tc_counts(ids)
    return _tc_epilogue(rcnt, psum, b.reshape(1, _NCLS))
